# baseline (speedup 1.0000x reference)
import jax
import jax.numpy as jnp
from jax import lax
from jax.experimental import pallas as pl
from jax.experimental.pallas import tpu as pltpu
from jax.experimental.pallas import tpu_sc as plsc

MAX_LEN = 4096
EMB = 128
BATCH = 4

NC = 2
NS = 16
NW = NC * NS
S_CHUNK = MAX_LEN // NW
ROWS_W = BATCH * S_CHUNK


def _sc_body(x_hbm, pos_hbm, idx_hbm, out_hbm, idx_v, pos_v, acc_sh):
    cid = lax.axis_index("c")
    sid = lax.axis_index("s")
    wid = sid * NC + cid
    s0 = wid * S_CHUNK
    base = sid * ROWS_W

    pltpu.sync_copy(idx_hbm.at[wid], idx_v)
    pltpu.sync_copy(pos_hbm.at[pl.ds(s0, S_CHUNK)], pos_v)
    for b in range(BATCH):
        pltpu.sync_copy(
            x_hbm.at[pl.ds(b * MAX_LEN + s0, S_CHUNK)],
            acc_sh.at[pl.ds(base + b * S_CHUNK, S_CHUNK)],
        )
    for b in range(BATCH):
        pltpu.sync_copy(pos_v, acc_sh.at[idx_v.at[b]], add=True)
    for b in range(BATCH):
        pltpu.sync_copy(
            acc_sh.at[pl.ds(base + b * S_CHUNK, S_CHUNK)],
            out_hbm.at[pl.ds(b * MAX_LEN + s0, S_CHUNK)],
        )


def kernel(x, pos_table):
    x2 = x.reshape(BATCH * MAX_LEN, EMB)
    sid_of_w = jnp.arange(NW) // NC
    idx = (
        sid_of_w[:, None, None] * ROWS_W
        + jnp.arange(BATCH)[None, :, None] * S_CHUNK
        + jnp.arange(S_CHUNK)[None, None, :]
    ).astype(jnp.int32)

    mesh = plsc.VectorSubcoreMesh(
        core_axis_name="c", subcore_axis_name="s", num_cores=NC, num_subcores=NS
    )
    out = pl.kernel(
        _sc_body,
        out_type=jax.ShapeDtypeStruct((BATCH * MAX_LEN, EMB), jnp.float32),
        mesh=mesh,
        scratch_types=[
            pltpu.VMEM((BATCH, S_CHUNK), jnp.int32),
            pltpu.VMEM((S_CHUNK, EMB), jnp.float32),
            pltpu.VMEM_SHARED((NS * ROWS_W, EMB), jnp.float32),
        ],
    )(x2, pos_table, idx)
    return out.reshape(BATCH, MAX_LEN, EMB)


# speedup vs baseline: 1.2142x; 1.2142x over previous
import jax
import jax.numpy as jnp
from jax import lax
from jax.experimental import pallas as pl
from jax.experimental.pallas import tpu as pltpu
from jax.experimental.pallas import tpu_sc as plsc

MAX_LEN = 4096
EMB = 128
BATCH = 4

NC = 2
NS = 16
NW = NC * NS
S_CHUNK = MAX_LEN // NW
ROWS_W = BATCH * S_CHUNK


def _sc_body(x_hbm, pos_hbm, idx_hbm, out_hbm, idx_v, pos_v, acc_sh,
             sems, out_sem):
    cid = lax.axis_index("c")
    sid = lax.axis_index("s")
    wid = sid * NC + cid
    s0 = wid * S_CHUNK
    base = sid * ROWS_W

    in_copies = [
        pltpu.async_copy(
            x_hbm.at[pl.ds(b * MAX_LEN + s0, S_CHUNK)],
            acc_sh.at[pl.ds(base + b * S_CHUNK, S_CHUNK)],
            sems.at[b],
        )
        for b in range(BATCH)
    ]
    pltpu.sync_copy(idx_hbm.at[sid], idx_v)
    pltpu.sync_copy(pos_hbm.at[pl.ds(s0, S_CHUNK)], pos_v)
    out_copies = []
    for b in range(BATCH):
        in_copies[b].wait()
        pltpu.sync_copy(pos_v, acc_sh.at[idx_v.at[b]], add=True)
        out_copies.append(
            pltpu.async_copy(
                acc_sh.at[pl.ds(base + b * S_CHUNK, S_CHUNK)],
                out_hbm.at[pl.ds(b * MAX_LEN + s0, S_CHUNK)],
                out_sem,
            )
        )
    for c in out_copies:
        c.wait()


def kernel(x, pos_table):
    x2 = x.reshape(BATCH * MAX_LEN, EMB)
    idx = (
        jnp.arange(NS)[:, None, None] * ROWS_W
        + jnp.arange(BATCH)[None, :, None] * S_CHUNK
        + jnp.arange(S_CHUNK)[None, None, :]
    ).astype(jnp.int32)

    mesh = plsc.VectorSubcoreMesh(
        core_axis_name="c", subcore_axis_name="s", num_cores=NC, num_subcores=NS
    )
    out = pl.kernel(
        _sc_body,
        out_type=jax.ShapeDtypeStruct((BATCH * MAX_LEN, EMB), jnp.float32),
        mesh=mesh,
        scratch_types=[
            pltpu.VMEM((BATCH, S_CHUNK), jnp.int32),
            pltpu.VMEM((S_CHUNK, EMB), jnp.float32),
            pltpu.VMEM_SHARED((NS * ROWS_W, EMB), jnp.float32),
            pltpu.SemaphoreType.DMA((BATCH,)),
            pltpu.SemaphoreType.DMA,
        ],
    )(x2, pos_table, idx)
    return out.reshape(BATCH, MAX_LEN, EMB)
